# baseline (speedup 1.0000x reference)
import functools
import math

import jax
import jax.numpy as jnp
from jax import lax
from jax.experimental import pallas as pl
from jax.experimental.pallas import tpu as pltpu
from jax.experimental.pallas import tpu_sc as plsc

D_MODEL = 64
DPAD = 128
SCALE = math.sqrt(D_MODEL)
NUM_WORKERS = 32
LANES = 16
BBLK = 128
SEQ = 200


def _transpose_scale(rows, outv):
    iota = lax.iota(jnp.int32, LANES)

    def d_body(d, c):
        cidx = jnp.full((LANES,), 0, jnp.int32) + d
        for g in range(BBLK // LANES):
            ridx = iota + (g * LANES)
            v = plsc.load_gather(rows, [ridx, cidx])
            outv[d, pl.ds(g * LANES, LANES)] = v * SCALE
        return c

    lax.fori_loop(0, D_MODEL, d_body, 0, unroll=2)


def _emb_kernel_body(xt_hbm, table_hbm, out_hbm,
                     idx_v, r0, r1, o0, o1,
                     gsem0, gsem1, wsem0, wsem1):
    wid = lax.axis_index("s") * 2 + lax.axis_index("c")
    col = wid * BBLK
    pltpu.sync_copy(xt_hbm.at[:, pl.ds(col, BBLK)], idx_v)

    def gather(s, buf, sem):
        pltpu.async_copy(table_hbm.at[idx_v.at[s]], buf, sem)

    def wait_gather(buf, sem):
        pltpu.make_async_copy(table_hbm.at[idx_v.at[0]], buf, sem).wait()

    def write(s, buf, sem):
        pltpu.async_copy(buf, out_hbm.at[s, :, pl.ds(col, BBLK)], sem)

    def wait_write(buf, sem):
        pltpu.make_async_copy(buf, out_hbm.at[0, :, pl.ds(col, BBLK)], sem).wait()

    gather(0, r0, gsem0)

    wait_gather(r0, gsem0)
    gather(1, r1, gsem1)
    _transpose_scale(r0, o0)
    write(0, o0, wsem0)

    wait_gather(r1, gsem1)
    gather(2, r0, gsem0)
    _transpose_scale(r1, o1)
    write(1, o1, wsem1)

    def pair_body(sp, carry):
        s = 2 * sp
        wait_gather(r0, gsem0)
        gather(s + 1, r1, gsem1)
        wait_write(o0, wsem0)
        _transpose_scale(r0, o0)
        write(s, o0, wsem0)

        wait_gather(r1, gsem1)
        gather(s + 2, r0, gsem0)
        wait_write(o1, wsem1)
        _transpose_scale(r1, o1)
        write(s + 1, o1, wsem1)
        return carry

    lax.fori_loop(1, SEQ // 2 - 1, pair_body, 0)

    s = SEQ - 2
    wait_gather(r0, gsem0)
    gather(s + 1, r1, gsem1)
    wait_write(o0, wsem0)
    _transpose_scale(r0, o0)
    write(s, o0, wsem0)

    wait_gather(r1, gsem1)
    wait_write(o1, wsem1)
    _transpose_scale(r1, o1)
    write(s + 1, o1, wsem1)

    wait_write(o0, wsem0)
    wait_write(o1, wsem1)


def kernel(x, table):
    b, seq = x.shape
    if x.dtype != jnp.int32:
        x = x.astype(jnp.int32)
    xt = jnp.swapaxes(x, 0, 1)
    tp = jnp.pad(table, ((0, 0), (0, DPAD - D_MODEL)))

    mesh = plsc.VectorSubcoreMesh(core_axis_name="c", subcore_axis_name="s")
    emb = functools.partial(
        pl.kernel,
        mesh=mesh,
        compiler_params=pltpu.CompilerParams(
            use_tc_tiling_on_sc=True, needs_layout_passes=False),
        out_type=jax.ShapeDtypeStruct((seq, D_MODEL, b), jnp.float32),
        scratch_types=[
            pltpu.VMEM((SEQ, BBLK), jnp.int32),
            pltpu.VMEM((BBLK, DPAD), jnp.float32),
            pltpu.VMEM((BBLK, DPAD), jnp.float32),
            pltpu.VMEM((D_MODEL, BBLK), jnp.float32),
            pltpu.VMEM((D_MODEL, BBLK), jnp.float32),
            pltpu.SemaphoreType.DMA,
            pltpu.SemaphoreType.DMA,
            pltpu.SemaphoreType.DMA,
            pltpu.SemaphoreType.DMA,
        ],
    )(_emb_kernel_body)

    out = emb(xt, tp)
    return jnp.transpose(out, (2, 0, 1))


# speedup vs baseline: 1.6613x; 1.6613x over previous
import functools
import math

import jax
import jax.numpy as jnp
from jax import lax
from jax.experimental import pallas as pl
from jax.experimental.pallas import tpu as pltpu
from jax.experimental.pallas import tpu_sc as plsc

D_MODEL = 64
DPAD = 128
SCALE = math.sqrt(D_MODEL)
NUM_WORKERS = 32
LANES = 16
BBLK = 128
SEQ = 200


def _make_diag_idx():
    iota = lax.iota(jnp.int32, LANES)
    rots = [lax.rem(iota + k, LANES) for k in range(LANES)]
    return iota, rots


def _transpose_scale(rows, outv, iota, rots):
    def g_body(g, c):
        ridx = iota + g * LANES
        for d0 in range(D_MODEL // LANES):
            for k in range(LANES):
                cidx = rots[k] + (d0 * LANES)
                v = plsc.load_gather(rows, [ridx, cidx])
                plsc.store_scatter(outv, [cidx, ridx], v * SCALE)
        return c

    lax.fori_loop(0, BBLK // LANES, g_body, 0)


def _emb_kernel_body(xt_hbm, table_hbm, out_hbm,
                     idx_v, r0, r1, o0, o1,
                     gsem0, gsem1, wsem0, wsem1):
    iota, rots = _make_diag_idx()
    wid = lax.axis_index("s") * 2 + lax.axis_index("c")
    col = wid * BBLK
    pltpu.sync_copy(xt_hbm.at[:, pl.ds(col, BBLK)], idx_v)

    def gather(s, buf, sem):
        pltpu.async_copy(table_hbm.at[idx_v.at[s]], buf, sem)

    def wait_gather(buf, sem):
        pltpu.make_async_copy(table_hbm.at[idx_v.at[0]], buf, sem).wait()

    def write(s, buf, sem):
        pltpu.async_copy(buf, out_hbm.at[s, :, pl.ds(col, BBLK)], sem)

    def wait_write(buf, sem):
        pltpu.make_async_copy(buf, out_hbm.at[0, :, pl.ds(col, BBLK)], sem).wait()

    gather(0, r0, gsem0)

    wait_gather(r0, gsem0)
    gather(1, r1, gsem1)
    _transpose_scale(r0, o0, iota, rots)
    write(0, o0, wsem0)

    wait_gather(r1, gsem1)
    gather(2, r0, gsem0)
    _transpose_scale(r1, o1, iota, rots)
    write(1, o1, wsem1)

    def pair_body(sp, carry):
        s = 2 * sp
        wait_gather(r0, gsem0)
        gather(s + 1, r1, gsem1)
        wait_write(o0, wsem0)
        _transpose_scale(r0, o0, iota, rots)
        write(s, o0, wsem0)

        wait_gather(r1, gsem1)
        gather(s + 2, r0, gsem0)
        wait_write(o1, wsem1)
        _transpose_scale(r1, o1, iota, rots)
        write(s + 1, o1, wsem1)
        return carry

    lax.fori_loop(1, SEQ // 2 - 1, pair_body, 0)

    s = SEQ - 2
    wait_gather(r0, gsem0)
    gather(s + 1, r1, gsem1)
    wait_write(o0, wsem0)
    _transpose_scale(r0, o0, iota, rots)
    write(s, o0, wsem0)

    wait_gather(r1, gsem1)
    wait_write(o1, wsem1)
    _transpose_scale(r1, o1, iota, rots)
    write(s + 1, o1, wsem1)

    wait_write(o0, wsem0)
    wait_write(o1, wsem1)


def kernel(x, table):
    b, seq = x.shape
    if x.dtype != jnp.int32:
        x = x.astype(jnp.int32)
    xt = jnp.swapaxes(x, 0, 1)
    tp = jnp.pad(table, ((0, 0), (0, DPAD - D_MODEL)))

    mesh = plsc.VectorSubcoreMesh(core_axis_name="c", subcore_axis_name="s")
    emb = functools.partial(
        pl.kernel,
        mesh=mesh,
        compiler_params=pltpu.CompilerParams(
            use_tc_tiling_on_sc=True, needs_layout_passes=False),
        out_type=jax.ShapeDtypeStruct((seq, D_MODEL, b), jnp.float32),
        scratch_types=[
            pltpu.VMEM((SEQ, BBLK), jnp.int32),
            pltpu.VMEM((BBLK, DPAD), jnp.float32),
            pltpu.VMEM((BBLK, DPAD), jnp.float32),
            pltpu.VMEM((D_MODEL, BBLK), jnp.float32),
            pltpu.VMEM((D_MODEL, BBLK), jnp.float32),
            pltpu.SemaphoreType.DMA,
            pltpu.SemaphoreType.DMA,
            pltpu.SemaphoreType.DMA,
            pltpu.SemaphoreType.DMA,
        ],
    )(_emb_kernel_body)

    out = emb(xt, tp)
    return jnp.transpose(out, (2, 0, 1))
